# Initial kernel scaffold; baseline (speedup 1.0000x reference)
import jax
import jax.numpy as jnp
from jax.experimental import pallas as pl


def kernel(x, edge_index, W1, b1, W2, b2, Wh, bh):
    raise NotImplementedError("write your pallas kernel here")


# speedup vs baseline: 8.3046x; 8.3046x over previous
import functools

import jax
import jax.numpy as jnp
from jax import lax
from jax.experimental import pallas as pl
from jax.experimental.pallas import tpu as pltpu
from jax.experimental.pallas import tpu_sc as plsc

N = 10000
NPAD = 10240
E = 160000
H = 128
NC, NS, LANES = 2, 16, 16
NW = NC * NS
CHUNK = 128
EPAD = 163840
CPT = EPAD // (NW * CHUNK)
RPS = NPAD // NS
TRASH = N + 16

_MESH = plsc.VectorSubcoreMesh(core_axis_name="c", subcore_axis_name="s")
_BLK = 1000


def _sc_degree(dst2d, zeros16, ones16):
    @functools.partial(
        pl.kernel,
        out_type=jax.ShapeDtypeStruct((NC, NPAD, LANES), jnp.float32),
        mesh=_MESH,
        scratch_types=[
            pltpu.VMEM((CPT, CHUNK), jnp.int32),
            pltpu.VMEM((CHUNK, LANES), jnp.float32),
            pltpu.VMEM_SHARED((NPAD, LANES), jnp.float32),
        ],
    )
    def deg_kernel(dst_hbm, z_hbm, one_hbm, out_hbm, idx_v, ones_v, shared):
        c = lax.axis_index("c")
        s = lax.axis_index("s")
        w = s * NC + c
        pltpu.sync_copy(z_hbm.at[pl.ds(s * RPS, RPS)],
                        shared.at[pl.ds(s * RPS, RPS)])
        pltpu.sync_copy(one_hbm, ones_v)
        pltpu.sync_copy(dst_hbm.at[pl.ds(w * CPT, CPT)], idx_v)
        plsc.subcore_barrier()

        @pl.loop(0, CPT)
        def _(j):
            pltpu.sync_copy(ones_v, shared.at[idx_v.at[j]], add=True)

        plsc.subcore_barrier()
        pltpu.sync_copy(shared.at[pl.ds(s * RPS, RPS)],
                        out_hbm.at[c, pl.ds(s * RPS, RPS)])

    return deg_kernel(dst2d, zeros16, ones16)


def _sc_scatter(g, src2d, dst2d, zeros128):
    @functools.partial(
        pl.kernel,
        out_type=jax.ShapeDtypeStruct((NC, NPAD, H), jnp.float32),
        mesh=_MESH,
        scratch_types=[
            pltpu.VMEM((CPT, CHUNK), jnp.int32),
            pltpu.VMEM((CPT, CHUNK), jnp.int32),
            pltpu.VMEM((CHUNK, H), jnp.float32),
            pltpu.VMEM_SHARED((NPAD, H), jnp.float32),
            pltpu.SemaphoreType.DMA,
        ],
    )
    def scat_kernel(g_hbm, src_hbm, dst_hbm, z_hbm, out_hbm,
                    sidx_v, didx_v, rows_v, shared, sem):
        c = lax.axis_index("c")
        s = lax.axis_index("s")
        w = s * NC + c
        pltpu.sync_copy(z_hbm.at[pl.ds(s * RPS, RPS)],
                        shared.at[pl.ds(s * RPS, RPS)])
        pltpu.sync_copy(src_hbm.at[pl.ds(w * CPT, CPT)], sidx_v)
        pltpu.sync_copy(dst_hbm.at[pl.ds(w * CPT, CPT)], didx_v)
        plsc.subcore_barrier()

        @pl.loop(0, CPT)
        def _(j):
            pltpu.async_copy(g_hbm.at[sidx_v.at[j]], rows_v, sem).wait()
            pltpu.sync_copy(rows_v, shared.at[didx_v.at[j]], add=True)

        plsc.subcore_barrier()
        pltpu.sync_copy(shared.at[pl.ds(s * RPS, RPS)],
                        out_hbm.at[c, pl.ds(s * RPS, RPS)])

    return scat_kernel(g, src2d, dst2d, zeros128)


def _tc_in_matmul(x, W1):
    def body(x_ref, w_ref, o_ref):
        o_ref[...] = jnp.dot(x_ref[...], w_ref[...],
                             preferred_element_type=jnp.float32)

    return pl.pallas_call(
        body,
        grid=(N // _BLK,),
        in_specs=[pl.BlockSpec((_BLK, 256), lambda i: (i, 0)),
                  pl.BlockSpec((256, H), lambda i: (0, 0))],
        out_specs=pl.BlockSpec((_BLK, H), lambda i: (i, 0)),
        out_shape=jax.ShapeDtypeStruct((N, H), jnp.float32),
    )(x, W1)


def _tc_prep(degp, h1):
    def body(d_ref, h_ref, g_ref, di_ref):
        deg = d_ref[0, :, 0:1] + d_ref[1, :, 0:1] + 1.0
        dinv = lax.rsqrt(deg)
        di_ref[...] = dinv
        g_ref[...] = h_ref[...] * dinv

    return pl.pallas_call(
        body,
        grid=(N // _BLK,),
        in_specs=[pl.BlockSpec((NC, _BLK, LANES), lambda i: (0, i, 0)),
                  pl.BlockSpec((_BLK, H), lambda i: (i, 0))],
        out_specs=[pl.BlockSpec((_BLK, H), lambda i: (i, 0)),
                   pl.BlockSpec((_BLK, 1), lambda i: (i, 0))],
        out_shape=[jax.ShapeDtypeStruct((N, H), jnp.float32),
                   jax.ShapeDtypeStruct((N, 1), jnp.float32)],
    )(degp, h1)


def _tc_layer(sp, g, dinv, b, W):
    def body(p_ref, g_ref, di_ref, b_ref, w_ref, o_ref):
        agg = p_ref[0] + p_ref[1] + g_ref[...]
        z = jnp.maximum(agg * di_ref[...] + b_ref[...], 0.0)
        h = jnp.dot(z, w_ref[...], preferred_element_type=jnp.float32)
        o_ref[...] = h * di_ref[...]

    return pl.pallas_call(
        body,
        grid=(N // _BLK,),
        in_specs=[pl.BlockSpec((NC, _BLK, H), lambda i: (0, i, 0)),
                  pl.BlockSpec((_BLK, H), lambda i: (i, 0)),
                  pl.BlockSpec((_BLK, 1), lambda i: (i, 0)),
                  pl.BlockSpec((1, H), lambda i: (0, 0)),
                  pl.BlockSpec((H, H), lambda i: (0, 0))],
        out_specs=pl.BlockSpec((_BLK, H), lambda i: (i, 0)),
        out_shape=jax.ShapeDtypeStruct((N, H), jnp.float32),
    )(sp, g, dinv, b.reshape(1, H), W)


def _tc_head(sp, g, dinv, b2, Wh, bh):
    def body(p_ref, g_ref, di_ref, b_ref, w_ref, bh_ref, o_ref):
        agg = p_ref[0] + p_ref[1] + g_ref[...]
        z = jnp.maximum(agg * di_ref[...] + b_ref[...], 0.0)
        o_ref[...] = jnp.dot(z, w_ref[...],
                             preferred_element_type=jnp.float32) + bh_ref[...]

    return pl.pallas_call(
        body,
        grid=(N // _BLK,),
        in_specs=[pl.BlockSpec((NC, _BLK, H), lambda i: (0, i, 0)),
                  pl.BlockSpec((_BLK, H), lambda i: (i, 0)),
                  pl.BlockSpec((_BLK, 1), lambda i: (i, 0)),
                  pl.BlockSpec((1, H), lambda i: (0, 0)),
                  pl.BlockSpec((H, 1), lambda i: (0, 0)),
                  pl.BlockSpec((1, 1), lambda i: (0, 0))],
        out_specs=pl.BlockSpec((_BLK, 1), lambda i: (i, 0)),
        out_shape=jax.ShapeDtypeStruct((N, 1), jnp.float32),
    )(sp, g, dinv, b2.reshape(1, H), Wh, bh.reshape(1, 1))


def kernel(x, edge_index, W1, b1, W2, b2, Wh, bh):
    src = edge_index[0]
    dst = edge_index[1]
    pad = EPAD - E
    src2d = jnp.concatenate(
        [src, jnp.zeros((pad,), src.dtype)]).reshape(EPAD // CHUNK, CHUNK)
    dst2d = jnp.concatenate(
        [dst, jnp.full((pad,), TRASH, dst.dtype)]).reshape(EPAD // CHUNK, CHUNK)
    zeros16 = jnp.zeros((NPAD, LANES), jnp.float32)
    ones16 = jnp.ones((CHUNK, LANES), jnp.float32)
    zeros128 = jnp.zeros((NPAD, H), jnp.float32)

    degp = _sc_degree(dst2d, zeros16, ones16)
    h1 = _tc_in_matmul(x, W1)
    g1, dinv = _tc_prep(degp, h1)
    s1 = _sc_scatter(g1, src2d, dst2d, zeros128)
    g2 = _tc_layer(s1, g1, dinv, b1, W2)
    s2 = _sc_scatter(g2, src2d, dst2d, zeros128)
    logits = _tc_head(s2, g2, dinv, b2, Wh, bh)
    return logits.reshape(N)


# speedup vs baseline: 9.3801x; 1.1295x over previous
import functools

import jax
import jax.numpy as jnp
from jax import lax
from jax.experimental import pallas as pl
from jax.experimental.pallas import tpu as pltpu
from jax.experimental.pallas import tpu_sc as plsc

N = 10000
NPAD = 10240
E = 160000
H = 128
NC, NS, LANES = 2, 16, 16
NW = NC * NS
CHUNK = 128
EPAD = 163840
CPT = EPAD // (NW * CHUNK)
RPS = NPAD // NS
TRASH = N + 16

_MESH = plsc.VectorSubcoreMesh(core_axis_name="c", subcore_axis_name="s")
_BLK = 1000


def _sc_degree(dst2d, zeros16, ones16):
    @functools.partial(
        pl.kernel,
        out_type=jax.ShapeDtypeStruct((NC, NPAD, LANES), jnp.float32),
        mesh=_MESH,
        scratch_types=[
            pltpu.VMEM((CPT, CHUNK), jnp.int32),
            pltpu.VMEM((CHUNK, LANES), jnp.float32),
            pltpu.VMEM_SHARED((NPAD, LANES), jnp.float32),
        ],
    )
    def deg_kernel(dst_hbm, z_hbm, one_hbm, out_hbm, idx_v, ones_v, shared):
        c = lax.axis_index("c")
        s = lax.axis_index("s")
        w = s * NC + c
        pltpu.sync_copy(z_hbm.at[pl.ds(s * RPS, RPS)],
                        shared.at[pl.ds(s * RPS, RPS)])
        pltpu.sync_copy(one_hbm, ones_v)
        pltpu.sync_copy(dst_hbm.at[pl.ds(w * CPT, CPT)], idx_v)
        plsc.subcore_barrier()

        @pl.loop(0, CPT)
        def _(j):
            pltpu.sync_copy(ones_v, shared.at[idx_v.at[j]], add=True)

        plsc.subcore_barrier()
        pltpu.sync_copy(shared.at[pl.ds(s * RPS, RPS)],
                        out_hbm.at[c, pl.ds(s * RPS, RPS)])

    return deg_kernel(dst2d, zeros16, ones16)


def _sc_scatter(g, src2d, dst2d, zeros128):
    @functools.partial(
        pl.kernel,
        out_type=jax.ShapeDtypeStruct((NC, NPAD, H), jnp.float32),
        mesh=_MESH,
        scratch_types=[
            pltpu.VMEM((CPT, CHUNK), jnp.int32),
            pltpu.VMEM((CPT, CHUNK), jnp.int32),
            pltpu.VMEM((CHUNK, H), jnp.float32),
            pltpu.VMEM((CHUNK, H), jnp.float32),
            pltpu.VMEM_SHARED((NPAD, H), jnp.float32),
            pltpu.SemaphoreType.DMA,
            pltpu.SemaphoreType.DMA,
        ],
    )
    def scat_kernel(g_hbm, src_hbm, dst_hbm, z_hbm, out_hbm,
                    sidx_v, didx_v, rows_a, rows_b, shared, sem_a, sem_b):
        c = lax.axis_index("c")
        s = lax.axis_index("s")
        w = s * NC + c
        pltpu.sync_copy(z_hbm.at[pl.ds(s * RPS, RPS)],
                        shared.at[pl.ds(s * RPS, RPS)])
        pltpu.sync_copy(src_hbm.at[pl.ds(w * CPT, CPT)], sidx_v)
        pltpu.sync_copy(dst_hbm.at[pl.ds(w * CPT, CPT)], didx_v)
        plsc.subcore_barrier()

        pltpu.async_copy(g_hbm.at[sidx_v.at[0]], rows_a, sem_a)
        pltpu.async_copy(g_hbm.at[sidx_v.at[1]], rows_b, sem_b)

        def _wait(buf, sem):
            pltpu.make_async_copy(z_hbm.at[pl.ds(0, CHUNK)], buf, sem).wait()

        @pl.loop(0, CPT - 2, step=2)
        def _(j):
            _wait(rows_a, sem_a)
            pltpu.sync_copy(rows_a, shared.at[didx_v.at[j]], add=True)
            pltpu.async_copy(g_hbm.at[sidx_v.at[j + 2]], rows_a, sem_a)
            _wait(rows_b, sem_b)
            pltpu.sync_copy(rows_b, shared.at[didx_v.at[j + 1]], add=True)
            pltpu.async_copy(g_hbm.at[sidx_v.at[j + 3]], rows_b, sem_b)

        _wait(rows_a, sem_a)
        pltpu.sync_copy(rows_a, shared.at[didx_v.at[CPT - 2]], add=True)
        _wait(rows_b, sem_b)
        pltpu.sync_copy(rows_b, shared.at[didx_v.at[CPT - 1]], add=True)

        plsc.subcore_barrier()
        pltpu.sync_copy(shared.at[pl.ds(s * RPS, RPS)],
                        out_hbm.at[c, pl.ds(s * RPS, RPS)])

    return scat_kernel(g, src2d, dst2d, zeros128)


def _tc_in_matmul(x, W1):
    def body(x_ref, w_ref, o_ref):
        o_ref[...] = jnp.dot(x_ref[...], w_ref[...],
                             preferred_element_type=jnp.float32)

    return pl.pallas_call(
        body,
        grid=(N // _BLK,),
        in_specs=[pl.BlockSpec((_BLK, 256), lambda i: (i, 0)),
                  pl.BlockSpec((256, H), lambda i: (0, 0))],
        out_specs=pl.BlockSpec((_BLK, H), lambda i: (i, 0)),
        out_shape=jax.ShapeDtypeStruct((N, H), jnp.float32),
    )(x, W1)


def _tc_prep(degp, h1):
    def body(d_ref, h_ref, g_ref, di_ref):
        deg = d_ref[0, :, 0:1] + d_ref[1, :, 0:1] + 1.0
        dinv = lax.rsqrt(deg)
        di_ref[...] = dinv
        g_ref[...] = h_ref[...] * dinv

    return pl.pallas_call(
        body,
        grid=(N // _BLK,),
        in_specs=[pl.BlockSpec((NC, _BLK, LANES), lambda i: (0, i, 0)),
                  pl.BlockSpec((_BLK, H), lambda i: (i, 0))],
        out_specs=[pl.BlockSpec((_BLK, H), lambda i: (i, 0)),
                   pl.BlockSpec((_BLK, 1), lambda i: (i, 0))],
        out_shape=[jax.ShapeDtypeStruct((N, H), jnp.float32),
                   jax.ShapeDtypeStruct((N, 1), jnp.float32)],
    )(degp, h1)


def _tc_layer(sp, g, dinv, b, W):
    def body(p_ref, g_ref, di_ref, b_ref, w_ref, o_ref):
        agg = p_ref[0] + p_ref[1] + g_ref[...]
        z = jnp.maximum(agg * di_ref[...] + b_ref[...], 0.0)
        h = jnp.dot(z, w_ref[...], preferred_element_type=jnp.float32)
        o_ref[...] = h * di_ref[...]

    return pl.pallas_call(
        body,
        grid=(N // _BLK,),
        in_specs=[pl.BlockSpec((NC, _BLK, H), lambda i: (0, i, 0)),
                  pl.BlockSpec((_BLK, H), lambda i: (i, 0)),
                  pl.BlockSpec((_BLK, 1), lambda i: (i, 0)),
                  pl.BlockSpec((1, H), lambda i: (0, 0)),
                  pl.BlockSpec((H, H), lambda i: (0, 0))],
        out_specs=pl.BlockSpec((_BLK, H), lambda i: (i, 0)),
        out_shape=jax.ShapeDtypeStruct((N, H), jnp.float32),
    )(sp, g, dinv, b.reshape(1, H), W)


def _tc_head(sp, g, dinv, b2, Wh, bh):
    def body(p_ref, g_ref, di_ref, b_ref, w_ref, bh_ref, o_ref):
        agg = p_ref[0] + p_ref[1] + g_ref[...]
        z = jnp.maximum(agg * di_ref[...] + b_ref[...], 0.0)
        o_ref[...] = jnp.dot(z, w_ref[...],
                             preferred_element_type=jnp.float32) + bh_ref[...]

    return pl.pallas_call(
        body,
        grid=(N // _BLK,),
        in_specs=[pl.BlockSpec((NC, _BLK, H), lambda i: (0, i, 0)),
                  pl.BlockSpec((_BLK, H), lambda i: (i, 0)),
                  pl.BlockSpec((_BLK, 1), lambda i: (i, 0)),
                  pl.BlockSpec((1, H), lambda i: (0, 0)),
                  pl.BlockSpec((H, 1), lambda i: (0, 0)),
                  pl.BlockSpec((1, 1), lambda i: (0, 0))],
        out_specs=pl.BlockSpec((_BLK, 1), lambda i: (i, 0)),
        out_shape=jax.ShapeDtypeStruct((N, 1), jnp.float32),
    )(sp, g, dinv, b2.reshape(1, H), Wh, bh.reshape(1, 1))


def kernel(x, edge_index, W1, b1, W2, b2, Wh, bh):
    src = edge_index[0]
    dst = edge_index[1]
    pad = EPAD - E
    src2d = jnp.concatenate(
        [src, jnp.zeros((pad,), src.dtype)]).reshape(EPAD // CHUNK, CHUNK)
    dst2d = jnp.concatenate(
        [dst, jnp.full((pad,), TRASH, dst.dtype)]).reshape(EPAD // CHUNK, CHUNK)
    zeros16 = jnp.zeros((NPAD, LANES), jnp.float32)
    ones16 = jnp.ones((CHUNK, LANES), jnp.float32)
    zeros128 = jnp.zeros((NPAD, H), jnp.float32)

    degp = _sc_degree(dst2d, zeros16, ones16)
    h1 = _tc_in_matmul(x, W1)
    g1, dinv = _tc_prep(degp, h1)
    s1 = _sc_scatter(g1, src2d, dst2d, zeros128)
    g2 = _tc_layer(s1, g1, dinv, b1, W2)
    s2 = _sc_scatter(g2, src2d, dst2d, zeros128)
    logits = _tc_head(s2, g2, dinv, b2, Wh, bh)
    return logits.reshape(N)
